# baseline (speedup 1.0000x reference)
import functools

import jax
import jax.numpy as jnp
from jax import lax
from jax.experimental import pallas as pl
from jax.experimental.pallas import tpu as pltpu
from jax.experimental.pallas import tpu_sc as plsc

N = 10000
E = 320000
G = 64
D1 = 336
D2 = 168
CW = 128
NCH1 = 3
NCH2 = 2
D1P = NCH1 * CW
D2P = NCH2 * CW
WD = 128
CHUNK = 128
NCHUNKS = E // CHUNK
NC = 2
NS = 16
QBLK = 4
NPADC = 60
NCHP = NCHUNKS + NPADC
NQ = NCHP // QBLK
N_ACC = N + 8
GROUPS = N // 8
GPER, GREM = divmod(GROUPS, NS)
SGRP = 13
SROWS = SGRP * 8
RBLK = 1000
NBLK = N // RBLK

_mesh = functools.partial(
    plsc.VectorSubcoreMesh, core_axis_name="c", subcore_axis_name="s",
    num_cores=NC, num_subcores=NS)


def _my_chunk_range(sid, per_sc, sc_chunk0, nsplit=NS):
  per, rem = divmod(per_sc, nsplit)
  n_my = per + jnp.where(sid < rem, 1, 0)
  base = sc_chunk0 + sid * per + jnp.minimum(sid, rem)
  return base, n_my, per + (1 if rem else 0)


def _tile_rows(s):
  base_g = s * GPER + jnp.minimum(s, GREM)
  return base_g * 8


def _copy_rows_out(s, acc_s, stage_v, write_fn):
  r0 = _tile_rows(s)
  for k in range(GPER // SGRP):
    rr = pl.multiple_of(r0 + k * SROWS, 8)
    pltpu.sync_copy(acc_s.at[pl.ds(rr, SROWS)], stage_v)
    write_fn(stage_v, rr, SROWS)

  @pl.when(s < GREM)
  def _():
    rr = pl.multiple_of(r0 + GPER * 8, 8)
    pltpu.sync_copy(acc_s.at[pl.ds(rr, 8)], stage_v.at[pl.ds(0, 8)])
    write_fn(stage_v.at[pl.ds(0, 8)], rr, 8)


def _zero_rows(s, acc_s, zstage_v):
  r0 = _tile_rows(s)
  for k in range(GPER // SGRP):
    rr = pl.multiple_of(r0 + k * SROWS, 8)
    pltpu.sync_copy(zstage_v, acc_s.at[pl.ds(rr, SROWS)])

  @pl.when(s < GREM)
  def _():
    rr = pl.multiple_of(r0 + GPER * 8, 8)
    pltpu.sync_copy(zstage_v.at[pl.ds(0, 8)], acc_s.at[pl.ds(rr, 8)])


def _deg_pipe(ones_v, dst3_hbm, acc_s, q0, npairs, didx, ssem):
  def stage(par, qb):
    pltpu.sync_copy(dst3_hbm.at[qb], didx.at[par])

  def sdesc(bb, par, k):
    return pltpu.make_async_copy(ones_v, acc_s.at[didx.at[par, k]], ssem[bb])

  def do_pair(qb0, first):
    for par in (0, 1):
      stage(par, qb0 + par)
      for k in range(QBLK):
        j = par * QBLK + k
        bb = j % 2
        if not (first and j < 2):
          sdesc(bb, 0, 0).wait()
        sdesc(bb, par, k).start(add=True)

  do_pair(q0, True)

  def pair_body(p, carry):
    do_pair(q0 + 2 * p, False)
    return carry

  lax.fori_loop(1, npairs, pair_body, 0)
  sdesc(0, 0, 0).wait()
  sdesc(1, 0, 0).wait()


def _deg_body(dst3_hbm, ones_hbm, zeros_hbm, out_hbm,
              didx, ones_v, zstage_v, acc_s, ss0, ss1):
  c = lax.axis_index("c")
  s = lax.axis_index("s")
  wid = c * NS + s
  pltpu.sync_copy(ones_hbm, ones_v)
  pltpu.sync_copy(zeros_hbm, zstage_v)
  _zero_rows(s, acc_s, zstage_v)
  plsc.subcore_barrier()

  nb = NQ // (NC * NS)
  _deg_pipe(ones_v, dst3_hbm, acc_s, wid * nb, nb // 2, didx, (ss0, ss1))
  plsc.subcore_barrier()

  def write_fn(stg, rr, nrows):
    pltpu.sync_copy(stg, out_hbm.at[c, pl.ds(rr, nrows)])

  _copy_rows_out(s, acc_s, zstage_v, write_fn)


def _deg_call(dst3, ones_arr, zeros_arr):
  return pl.kernel(
      _deg_body,
      out_type=jax.ShapeDtypeStruct((NC, N, WD), jnp.float32),
      mesh=_mesh(),
      scratch_types=[
          pltpu.VMEM((2, QBLK, CHUNK), jnp.int32),
          pltpu.VMEM((CHUNK, WD), jnp.float32),
          pltpu.VMEM((SROWS, WD), jnp.float32),
          pltpu.VMEM_SHARED((N_ACC, WD), jnp.float32),
          pltpu.SemaphoreType.DMA,
          pltpu.SemaphoreType.DMA,
      ],
      name="sc_deg_hist",
  )(dst3, ones_arr, zeros_arr)


def _mp_pipe(h_hbm, src3, dst3, acc_s, q0, nblocks, sidx, didx, rows):
  def block_body(b, carry):
    qb = q0 + b
    pltpu.sync_copy(src3.at[qb], sidx)
    pltpu.sync_copy(dst3.at[qb], didx)
    for t in range(2):
      si = sidx.at[pl.ds(t * 2 * CHUNK, 2 * CHUNK)]
      di = didx.at[pl.ds(t * 2 * CHUNK, 2 * CHUNK)]
      pltpu.sync_copy(h_hbm.at[si], rows)
      pltpu.sync_copy(rows, acc_s.at[di], add=True)
    return carry

  lax.fori_loop(0, nblocks, block_body, 0)


def _mp_body(nsrc, hsel, q0_fn, nblocks, *refs):
  h_hbms = refs[:nsrc]
  src3, dst3, zeros_hbm, out_hbm = refs[nsrc:nsrc + 4]
  (sidx, didx, rows, zstage_v, acc_s) = refs[nsrc + 4:]
  c = lax.axis_index("c")
  s = lax.axis_index("s")
  pltpu.sync_copy(zeros_hbm, zstage_v)
  _zero_rows(s, acc_s, zstage_v)
  plsc.subcore_barrier()

  for cc in range(NC):
    @pl.when(c == cc)
    def _(cc=cc):
      _mp_pipe(h_hbms[hsel[cc]], src3, dst3, acc_s, q0_fn(cc, s), nblocks,
               sidx, didx, rows)

  plsc.subcore_barrier()

  def write_fn(stg, rr, nrows):
    pltpu.sync_copy(stg, out_hbm.at[c, pl.ds(rr, nrows)])

  _copy_rows_out(s, acc_s, zstage_v, write_fn)


def _mp_kernel(nsrc, hsel, q0_fn, nblocks, name, h_arrs, src3, dst3,
               zeros_arr):
  return pl.kernel(
      functools.partial(_mp_body, nsrc, hsel, q0_fn, nblocks),
      out_type=jax.ShapeDtypeStruct((NC, N, CW), jnp.float32),
      mesh=_mesh(),
      scratch_types=[
          pltpu.VMEM((4 * CHUNK,), jnp.int32),
          pltpu.VMEM((4 * CHUNK,), jnp.int32),
          pltpu.VMEM((2 * CHUNK, CW), jnp.float32),
          pltpu.VMEM((SROWS, CW), jnp.float32),
          pltpu.VMEM_SHARED((N_ACC, CW), jnp.float32),
      ],
      name=name,
  )(*h_arrs, src3, dst3, zeros_arr)


def _mp1_call(src3_mp1, dst3_mp1, hs_list, zeros_arr):
  nb = NQ // NC // NS
  outs = []
  for j in range(NCH1):
    hs_j = hs_list[j]
    if outs:
      hs_j, _ = lax.optimization_barrier((hs_j, outs[-1]))
    outs.append(_mp_kernel(
        1, (0, 0), lambda cc, s: cc * (NQ // NC) + s * nb, nb,
        f"sc_mp1_c{j}", (hs_j,), src3_mp1, dst3_mp1, zeros_arr))
  return outs


def _mp2_call(src3, dst3, hs_list, zeros_arr):
  nb = NQ // NS
  return _mp_kernel(
      2, (0, 1), lambda cc, s: s * nb, nb,
      "sc_mp2", tuple(hs_list), src3, dst3, zeros_arr)


def _dis_block(dp_ref):
  deg = dp_ref[0, :, 0:1] + dp_ref[1, :, 0:1] + 1.0
  return lax.rsqrt(deg)


def _tc1_body(x_ref, w1_ref, dp_ref, hsa_ref, hsb_ref, hsc_ref):
  dis = _dis_block(dp_ref)
  h = lax.dot_general(x_ref[...], w1_ref[...], (((1,), (1,)), ((), ())),
                      preferred_element_type=jnp.float32)
  hs = h * dis
  hsa_ref[...] = hs[:, :CW]
  hsb_ref[...] = hs[:, CW:2 * CW]
  hsc_ref[...] = hs[:, 2 * CW:]


def _tc1_call(x, W1p, degparts):
  return pl.pallas_call(
      _tc1_body,
      grid=(NBLK,),
      in_specs=[
          pl.BlockSpec((RBLK, 128), lambda i: (i, 0)),
          pl.BlockSpec((D1P, 128), lambda i: (0, 0)),
          pl.BlockSpec((NC, RBLK, WD), lambda i: (0, i, 0)),
      ],
      out_specs=[
          pl.BlockSpec((RBLK, CW), lambda i: (i, 0)),
          pl.BlockSpec((RBLK, CW), lambda i: (i, 0)),
          pl.BlockSpec((RBLK, CW), lambda i: (i, 0)),
      ],
      out_shape=[
          jax.ShapeDtypeStruct((N, CW), jnp.float32),
          jax.ShapeDtypeStruct((N, CW), jnp.float32),
          jax.ShapeDtypeStruct((N, CW), jnp.float32),
      ],
      name="tc1_matmul_scale",
  )(x, W1p, degparts)


def _tc2_body(ms0_ref, ms1_ref, ms2_ref, hsa_ref, hsb_ref, hsc_ref,
              dp_ref, b1_ref, w2_ref, hs2a_ref, hs2b_ref):
  dis = _dis_block(dp_ref)
  msf = jnp.concatenate(
      [ms0_ref[0] + ms0_ref[1], ms1_ref[0] + ms1_ref[1],
       ms2_ref[0] + ms2_ref[1]], axis=1)[:, :D1]
  hsf = jnp.concatenate([hsa_ref[...], hsb_ref[...], hsc_ref[...]],
                        axis=1)[:, :D1]
  h1 = jax.nn.relu(dis * (msf + hsf) + b1_ref[...])
  h2 = lax.dot_general(h1, w2_ref[...], (((1,), (1,)), ((), ())),
                       preferred_element_type=jnp.float32)
  hs2 = h2 * dis
  hs2a_ref[...] = hs2[:, :CW]
  hs2b_ref[...] = jnp.concatenate(
      [hs2[:, CW:], jnp.zeros((RBLK, D2P - D2), jnp.float32)], axis=1)


def _tc2_call(msgsum1, hs_list, degparts, b1r, W2):
  return pl.pallas_call(
      _tc2_body,
      grid=(NBLK,),
      in_specs=[
          pl.BlockSpec((NC, RBLK, CW), lambda i: (0, i, 0)),
          pl.BlockSpec((NC, RBLK, CW), lambda i: (0, i, 0)),
          pl.BlockSpec((NC, RBLK, CW), lambda i: (0, i, 0)),
          pl.BlockSpec((RBLK, CW), lambda i: (i, 0)),
          pl.BlockSpec((RBLK, CW), lambda i: (i, 0)),
          pl.BlockSpec((RBLK, CW), lambda i: (i, 0)),
          pl.BlockSpec((NC, RBLK, WD), lambda i: (0, i, 0)),
          pl.BlockSpec((1, D1), lambda i: (0, 0)),
          pl.BlockSpec((D2, D1), lambda i: (0, 0)),
      ],
      out_specs=[
          pl.BlockSpec((RBLK, CW), lambda i: (i, 0)),
          pl.BlockSpec((RBLK, CW), lambda i: (i, 0)),
      ],
      out_shape=[
          jax.ShapeDtypeStruct((N, CW), jnp.float32),
          jax.ShapeDtypeStruct((N, CW), jnp.float32),
      ],
      name="tc2_update_matmul",
  )(*msgsum1, *hs_list, degparts, b1r, W2)


def _tc3_body(ms_ref, hs2a_ref, hs2b_ref, dp_ref, b2_ref, batch_ref,
              wg_ref, bg_ref, wf_ref, bf_ref, wo_ref, bo_ref,
              o_ref, acc_ref):
  i = pl.program_id(0)

  @pl.when(i == 0)
  def _():
    acc_ref[...] = jnp.full((G, D2), -jnp.inf, dtype=jnp.float32)

  dis = _dis_block(dp_ref)
  msf = jnp.concatenate([ms_ref[0], ms_ref[1]], axis=1)[:, :D2]
  hsf = jnp.concatenate([hs2a_ref[...], hs2b_ref[...]], axis=1)[:, :D2]
  h = jax.nn.relu(dis * (msf + hsf) + b2_ref[...])
  b = batch_ref[...]
  glo = jnp.min(b)
  ghi = jnp.max(b)

  def seg_body(g, carry):
    m = (b == g)
    v = jnp.max(jnp.where(m, h, -jnp.inf), axis=0, keepdims=True)
    acc_ref[pl.ds(g, 1), :] = jnp.maximum(acc_ref[pl.ds(g, 1), :], v)
    return carry

  lax.fori_loop(glo, ghi + 1, seg_body, 0)

  @pl.when(i == NBLK - 1)
  def _():
    g0 = acc_ref[...]
    g1 = jax.nn.relu(
        lax.dot_general(g0, wg_ref[...], (((1,), (1,)), ((), ())),
                        preferred_element_type=jnp.float32) + bg_ref[...])
    g2 = jax.nn.relu(
        lax.dot_general(g1, wf_ref[...], (((1,), (1,)), ((), ())),
                        preferred_element_type=jnp.float32) + bf_ref[...])
    res = lax.dot_general(
        g2, wo_ref[...], (((1,), (1,)), ((), ())),
        preferred_element_type=jnp.float32)
    o_ref[...] = res[:, 0:1] + bo_ref[0, 0]


def _tc3_call(msgsum2, hs2_list, degparts, b2r, batch2,
              Wg, bgr, Wf, bfr, Wo, bor):
  return pl.pallas_call(
      _tc3_body,
      grid=(NBLK,),
      in_specs=[
          pl.BlockSpec((NCH2, RBLK, CW), lambda i: (0, i, 0)),
          pl.BlockSpec((RBLK, CW), lambda i: (i, 0)),
          pl.BlockSpec((RBLK, CW), lambda i: (i, 0)),
          pl.BlockSpec((NC, RBLK, WD), lambda i: (0, i, 0)),
          pl.BlockSpec((1, D2), lambda i: (0, 0)),
          pl.BlockSpec((RBLK, 1), lambda i: (i, 0)),
          pl.BlockSpec((84, D2), lambda i: (0, 0)),
          pl.BlockSpec((1, 84), lambda i: (0, 0)),
          pl.BlockSpec((42, 84), lambda i: (0, 0)),
          pl.BlockSpec((1, 42), lambda i: (0, 0)),
          pl.BlockSpec((8, 42), lambda i: (0, 0)),
          pl.BlockSpec((1, 1), lambda i: (0, 0)),
      ],
      out_specs=pl.BlockSpec((G, 1), lambda i: (0, 0)),
      out_shape=jax.ShapeDtypeStruct((G, 1), jnp.float32),
      scratch_shapes=[pltpu.VMEM((G, D2), jnp.float32)],
      name="tc3_pool_mlp",
  )(msgsum2, *hs2_list, degparts, b2r, batch2,
    Wg, bgr, Wf, bfr, Wo, bor)


def kernel(x, edge_index, batch, W1, b1, W2, b2, Wg, bg, Wf, bf, Wo, bo):
  src1d = edge_index[0]
  dst1d = edge_index[1]
  pad_s = jnp.zeros(((NPADC // 2) * CHUNK,), src1d.dtype)
  pad_d = jnp.full(((NPADC // 2) * CHUNK,), N, dst1d.dtype)
  dst3_deg = jnp.concatenate([dst1d, pad_d, pad_d]).reshape(NQ, QBLK, CHUNK)
  src3 = jnp.concatenate([src1d, pad_s, pad_s]).reshape(NQ, 4 * CHUNK)
  dst3 = jnp.concatenate([dst1d, pad_d, pad_d]).reshape(NQ, 4 * CHUNK)
  h0 = (NCHUNKS // NC) * CHUNK
  src3_mp1 = jnp.concatenate(
      [src1d[:h0], pad_s, src1d[h0:], pad_s]).reshape(NQ, 4 * CHUNK)
  dst3_mp1 = jnp.concatenate(
      [dst1d[:h0], pad_d, dst1d[h0:], pad_d]).reshape(NQ, 4 * CHUNK)
  W1p = jnp.pad(W1, ((0, D1P - D1), (0, 0)))
  oneswd = jnp.ones((CHUNK, WD), jnp.float32)
  zeroswd = jnp.zeros((SROWS, WD), jnp.float32)
  zeroscw = jnp.zeros((SROWS, CW), jnp.float32)
  batch2 = batch.reshape(N, 1)
  b1r = b1.reshape(1, D1)
  b2r = b2.reshape(1, D2)
  bgr = bg.reshape(1, 84)
  bfr = bf.reshape(1, 42)
  bor = bo.reshape(1, 1)

  Wop = jnp.pad(Wo, ((0, 7), (0, 0)))
  degparts = _deg_call(dst3_deg, oneswd, zeroswd)
  hs_list = _tc1_call(x, W1p, degparts)
  msgsum1 = _mp1_call(src3_mp1, dst3_mp1, hs_list, zeroscw)
  hs2_list = _tc2_call(msgsum1, hs_list, degparts, b1r, W2)
  msgsum2 = _mp2_call(src3, dst3, hs2_list, zeroscw)
  return _tc3_call(msgsum2, hs2_list, degparts, b2r, batch2,
                   Wg, bgr, Wf, bfr, Wop, bor)


# speedup vs baseline: 1.8905x; 1.8905x over previous
import functools

import jax
import jax.numpy as jnp
from jax import lax
from jax.experimental import pallas as pl
from jax.experimental.pallas import tpu as pltpu
from jax.experimental.pallas import tpu_sc as plsc

N = 10000
E = 320000
G = 64
D1 = 336
D2 = 168
CW = 128
NCH1 = 3
NCH2 = 2
D1P = NCH1 * CW
D2P = NCH2 * CW
WD = 128
CHUNK = 128
NCHUNKS = E // CHUNK
NC = 2
NS = 16
TBLK = 256
NT = E // TBLK
GROUPS = N // 8
GPER, GREM = divmod(GROUPS, NS)
SGRP = 13
SROWS = SGRP * 8
RBLK = 1000
NBLK = N // RBLK

_mesh = functools.partial(
    plsc.VectorSubcoreMesh, core_axis_name="c", subcore_axis_name="s",
    num_cores=NC, num_subcores=NS)


def _my_chunk_range(sid, per_sc, sc_chunk0, nsplit=NS):
  per, rem = divmod(per_sc, nsplit)
  n_my = per + jnp.where(sid < rem, 1, 0)
  base = sc_chunk0 + sid * per + jnp.minimum(sid, rem)
  return base, n_my, per + (1 if rem else 0)


def _tile_rows(s):
  base_g = s * GPER + jnp.minimum(s, GREM)
  return base_g * 8


def _copy_rows_out(s, acc_s, stage_v, write_fn):
  r0 = _tile_rows(s)
  for k in range(GPER // SGRP):
    rr = pl.multiple_of(r0 + k * SROWS, 8)
    pltpu.sync_copy(acc_s.at[pl.ds(rr, SROWS)], stage_v)
    write_fn(stage_v, rr, SROWS)

  @pl.when(s < GREM)
  def _():
    rr = pl.multiple_of(r0 + GPER * 8, 8)
    pltpu.sync_copy(acc_s.at[pl.ds(rr, 8)], stage_v.at[pl.ds(0, 8)])
    write_fn(stage_v.at[pl.ds(0, 8)], rr, 8)


def _zero_rows(s, acc_s, zstage_v):
  r0 = _tile_rows(s)
  for k in range(GPER // SGRP):
    rr = pl.multiple_of(r0 + k * SROWS, 8)
    pltpu.sync_copy(zstage_v, acc_s.at[pl.ds(rr, SROWS)])

  @pl.when(s < GREM)
  def _():
    rr = pl.multiple_of(r0 + GPER * 8, 8)
    pltpu.sync_copy(zstage_v.at[pl.ds(0, 8)], acc_s.at[pl.ds(rr, 8)])


def _deg_body(dst2_hbm, ones_hbm, zeros_hbm, out_hbm,
              idx_v, ones_v, zstage_v, acc_s):
  c = lax.axis_index("c")
  s = lax.axis_index("s")
  wid = c * NS + s
  pltpu.sync_copy(ones_hbm, ones_v)
  pltpu.sync_copy(zeros_hbm, zstage_v)
  _zero_rows(s, acc_s, zstage_v)
  plsc.subcore_barrier()

  per, rem = divmod(NT, NC * NS)
  n_my = per + jnp.where(wid < rem, 1, 0)
  base = wid * per + jnp.minimum(wid, rem)

  def chunk_body(i, carry):
    @pl.when(i < n_my)
    def _():
      pltpu.sync_copy(dst2_hbm.at[base + i], idx_v)
      pltpu.sync_copy(ones_v, acc_s.at[idx_v], add=True)
    return carry

  lax.fori_loop(0, per + (1 if rem else 0), chunk_body, 0)
  plsc.subcore_barrier()

  def write_fn(stg, rr, nrows):
    pltpu.sync_copy(stg, out_hbm.at[c, pl.ds(rr, nrows)])

  _copy_rows_out(s, acc_s, zstage_v, write_fn)


def _deg_call(dst2, ones_arr, zeros_arr):
  return pl.kernel(
      _deg_body,
      out_type=jax.ShapeDtypeStruct((NC, N, WD), jnp.float32),
      mesh=_mesh(),
      scratch_types=[
          pltpu.VMEM((TBLK,), jnp.int32),
          pltpu.VMEM((TBLK, WD), jnp.float32),
          pltpu.VMEM((SROWS, WD), jnp.float32),
          pltpu.VMEM_SHARED((N, WD), jnp.float32),
      ],
      name="sc_deg_hist",
  )(dst2, ones_arr, zeros_arr)


def _mp_pipe(h_hbm, src2, dst2, acc_s, base, n_my, nmax, sidx, didx, rows):
  def t_body(i, carry):
    @pl.when(i < n_my)
    def _():
      pltpu.sync_copy(src2.at[base + i], sidx)
      pltpu.sync_copy(dst2.at[base + i], didx)
      pltpu.sync_copy(h_hbm.at[sidx], rows)
      pltpu.sync_copy(rows, acc_s.at[didx], add=True)
    return carry

  lax.fori_loop(0, nmax, t_body, 0)


def _mp_body(nsrc, hsel, split_fn, *refs):
  h_hbms = refs[:nsrc]
  src2, dst2, zeros_hbm, out_hbm = refs[nsrc:nsrc + 4]
  (sidx, didx, rows, zstage_v, acc_s) = refs[nsrc + 4:]
  c = lax.axis_index("c")
  s = lax.axis_index("s")
  pltpu.sync_copy(zeros_hbm, zstage_v)
  _zero_rows(s, acc_s, zstage_v)
  plsc.subcore_barrier()

  for cc in range(NC):
    @pl.when(c == cc)
    def _(cc=cc):
      base, n_my, nmax = split_fn(cc, s)
      _mp_pipe(h_hbms[hsel[cc]], src2, dst2, acc_s, base, n_my, nmax,
               sidx, didx, rows)

  plsc.subcore_barrier()

  def write_fn(stg, rr, nrows):
    pltpu.sync_copy(stg, out_hbm.at[c, pl.ds(rr, nrows)])

  _copy_rows_out(s, acc_s, zstage_v, write_fn)


def _mp_kernel(nsrc, hsel, split_fn, name, h_arrs, src2, dst2, zeros_arr):
  return pl.kernel(
      functools.partial(_mp_body, nsrc, hsel, split_fn),
      out_type=jax.ShapeDtypeStruct((NC, N, CW), jnp.float32),
      mesh=_mesh(),
      scratch_types=[
          pltpu.VMEM((TBLK,), jnp.int32),
          pltpu.VMEM((TBLK,), jnp.int32),
          pltpu.VMEM((TBLK, CW), jnp.float32),
          pltpu.VMEM((SROWS, CW), jnp.float32),
          pltpu.VMEM_SHARED((N, CW), jnp.float32),
      ],
      name=name,
  )(*h_arrs, src2, dst2, zeros_arr)


def _split_half(cc, s):
  per, rem = divmod(NT // NC, NS)
  n_my = per + jnp.where(s < rem, 1, 0)
  base = cc * (NT // NC) + s * per + jnp.minimum(s, rem)
  return base, n_my, per + (1 if rem else 0)


def _split_full(cc, s):
  per, rem = divmod(NT, NS)
  n_my = per + jnp.where(s < rem, 1, 0)
  base = s * per + jnp.minimum(s, rem)
  return base, n_my, per + (1 if rem else 0)


def _mp1_call(src2, dst2, hs_list, zeros_arr):
  outs = []
  for j in range(NCH1):
    hs_j = hs_list[j]
    if outs:
      hs_j, _ = lax.optimization_barrier((hs_j, outs[-1]))
    outs.append(_mp_kernel(1, (0, 0), _split_half, f"sc_mp1_c{j}",
                           (hs_j,), src2, dst2, zeros_arr))
  return outs


def _mp2_call(src2, dst2, hs_list, zeros_arr):
  return _mp_kernel(2, (0, 1), _split_full, "sc_mp2",
                    tuple(hs_list), src2, dst2, zeros_arr)


def _dis_block(dp_ref):
  deg = dp_ref[0, :, 0:1] + dp_ref[1, :, 0:1] + 1.0
  return lax.rsqrt(deg)


def _tc1_body(x_ref, w1_ref, dp_ref, hsa_ref, hsb_ref, hsc_ref):
  dis = _dis_block(dp_ref)
  h = lax.dot_general(x_ref[...], w1_ref[...], (((1,), (1,)), ((), ())),
                      preferred_element_type=jnp.float32)
  hs = h * dis
  hsa_ref[...] = hs[:, :CW]
  hsb_ref[...] = hs[:, CW:2 * CW]
  hsc_ref[...] = hs[:, 2 * CW:]


def _tc1_call(x, W1p, degparts):
  return pl.pallas_call(
      _tc1_body,
      grid=(NBLK,),
      in_specs=[
          pl.BlockSpec((RBLK, 128), lambda i: (i, 0)),
          pl.BlockSpec((D1P, 128), lambda i: (0, 0)),
          pl.BlockSpec((NC, RBLK, WD), lambda i: (0, i, 0)),
      ],
      out_specs=[
          pl.BlockSpec((RBLK, CW), lambda i: (i, 0)),
          pl.BlockSpec((RBLK, CW), lambda i: (i, 0)),
          pl.BlockSpec((RBLK, CW), lambda i: (i, 0)),
      ],
      out_shape=[
          jax.ShapeDtypeStruct((N, CW), jnp.float32),
          jax.ShapeDtypeStruct((N, CW), jnp.float32),
          jax.ShapeDtypeStruct((N, CW), jnp.float32),
      ],
      name="tc1_matmul_scale",
  )(x, W1p, degparts)


def _tc2_body(ms0_ref, ms1_ref, ms2_ref, hsa_ref, hsb_ref, hsc_ref,
              dp_ref, b1_ref, w2_ref, hs2a_ref, hs2b_ref):
  dis = _dis_block(dp_ref)
  msf = jnp.concatenate(
      [ms0_ref[0] + ms0_ref[1], ms1_ref[0] + ms1_ref[1],
       ms2_ref[0] + ms2_ref[1]], axis=1)[:, :D1]
  hsf = jnp.concatenate([hsa_ref[...], hsb_ref[...], hsc_ref[...]],
                        axis=1)[:, :D1]
  h1 = jax.nn.relu(dis * (msf + hsf) + b1_ref[...])
  h2 = lax.dot_general(h1, w2_ref[...], (((1,), (1,)), ((), ())),
                       preferred_element_type=jnp.float32)
  hs2 = h2 * dis
  hs2a_ref[...] = hs2[:, :CW]
  hs2b_ref[...] = jnp.concatenate(
      [hs2[:, CW:], jnp.zeros((RBLK, D2P - D2), jnp.float32)], axis=1)


def _tc2_call(msgsum1, hs_list, degparts, b1r, W2):
  return pl.pallas_call(
      _tc2_body,
      grid=(NBLK,),
      in_specs=[
          pl.BlockSpec((NC, RBLK, CW), lambda i: (0, i, 0)),
          pl.BlockSpec((NC, RBLK, CW), lambda i: (0, i, 0)),
          pl.BlockSpec((NC, RBLK, CW), lambda i: (0, i, 0)),
          pl.BlockSpec((RBLK, CW), lambda i: (i, 0)),
          pl.BlockSpec((RBLK, CW), lambda i: (i, 0)),
          pl.BlockSpec((RBLK, CW), lambda i: (i, 0)),
          pl.BlockSpec((NC, RBLK, WD), lambda i: (0, i, 0)),
          pl.BlockSpec((1, D1), lambda i: (0, 0)),
          pl.BlockSpec((D2, D1), lambda i: (0, 0)),
      ],
      out_specs=[
          pl.BlockSpec((RBLK, CW), lambda i: (i, 0)),
          pl.BlockSpec((RBLK, CW), lambda i: (i, 0)),
      ],
      out_shape=[
          jax.ShapeDtypeStruct((N, CW), jnp.float32),
          jax.ShapeDtypeStruct((N, CW), jnp.float32),
      ],
      name="tc2_update_matmul",
  )(*msgsum1, *hs_list, degparts, b1r, W2)


def _tc3_body(ms_ref, hs2a_ref, hs2b_ref, dp_ref, b2_ref, batch_ref,
              wg_ref, bg_ref, wf_ref, bf_ref, wo_ref, bo_ref,
              o_ref, acc_ref):
  i = pl.program_id(0)

  @pl.when(i == 0)
  def _():
    acc_ref[...] = jnp.full((G, D2), -jnp.inf, dtype=jnp.float32)

  dis = _dis_block(dp_ref)
  msf = jnp.concatenate([ms_ref[0], ms_ref[1]], axis=1)[:, :D2]
  hsf = jnp.concatenate([hs2a_ref[...], hs2b_ref[...]], axis=1)[:, :D2]
  h = jax.nn.relu(dis * (msf + hsf) + b2_ref[...])
  b = batch_ref[...]
  glo = jnp.min(b)
  ghi = jnp.max(b)

  def seg_body(g, carry):
    m = (b == g)
    v = jnp.max(jnp.where(m, h, -jnp.inf), axis=0, keepdims=True)
    acc_ref[pl.ds(g, 1), :] = jnp.maximum(acc_ref[pl.ds(g, 1), :], v)
    return carry

  lax.fori_loop(glo, ghi + 1, seg_body, 0)

  @pl.when(i == NBLK - 1)
  def _():
    g0 = acc_ref[...]
    g1 = jax.nn.relu(
        lax.dot_general(g0, wg_ref[...], (((1,), (1,)), ((), ())),
                        preferred_element_type=jnp.float32) + bg_ref[...])
    g2 = jax.nn.relu(
        lax.dot_general(g1, wf_ref[...], (((1,), (1,)), ((), ())),
                        preferred_element_type=jnp.float32) + bf_ref[...])
    res = lax.dot_general(
        g2, wo_ref[...], (((1,), (1,)), ((), ())),
        preferred_element_type=jnp.float32)
    o_ref[...] = res[:, 0:1] + bo_ref[0, 0]


def _tc3_call(msgsum2, hs2_list, degparts, b2r, batch2,
              Wg, bgr, Wf, bfr, Wo, bor):
  return pl.pallas_call(
      _tc3_body,
      grid=(NBLK,),
      in_specs=[
          pl.BlockSpec((NCH2, RBLK, CW), lambda i: (0, i, 0)),
          pl.BlockSpec((RBLK, CW), lambda i: (i, 0)),
          pl.BlockSpec((RBLK, CW), lambda i: (i, 0)),
          pl.BlockSpec((NC, RBLK, WD), lambda i: (0, i, 0)),
          pl.BlockSpec((1, D2), lambda i: (0, 0)),
          pl.BlockSpec((RBLK, 1), lambda i: (i, 0)),
          pl.BlockSpec((84, D2), lambda i: (0, 0)),
          pl.BlockSpec((1, 84), lambda i: (0, 0)),
          pl.BlockSpec((42, 84), lambda i: (0, 0)),
          pl.BlockSpec((1, 42), lambda i: (0, 0)),
          pl.BlockSpec((8, 42), lambda i: (0, 0)),
          pl.BlockSpec((1, 1), lambda i: (0, 0)),
      ],
      out_specs=pl.BlockSpec((G, 1), lambda i: (0, 0)),
      out_shape=jax.ShapeDtypeStruct((G, 1), jnp.float32),
      scratch_shapes=[pltpu.VMEM((G, D2), jnp.float32)],
      name="tc3_pool_mlp",
  )(msgsum2, *hs2_list, degparts, b2r, batch2,
    Wg, bgr, Wf, bfr, Wo, bor)


def kernel(x, edge_index, batch, W1, b1, W2, b2, Wg, bg, Wf, bf, Wo, bo):
  src1d = edge_index[0]
  dst1d = edge_index[1]
  src2 = src1d.reshape(NT, TBLK)
  dst2 = dst1d.reshape(NT, TBLK)
  W1p = jnp.pad(W1, ((0, D1P - D1), (0, 0)))
  oneswd = jnp.ones((TBLK, WD), jnp.float32)
  zeroswd = jnp.zeros((SROWS, WD), jnp.float32)
  zeroscw = jnp.zeros((SROWS, CW), jnp.float32)
  batch2 = batch.reshape(N, 1)
  b1r = b1.reshape(1, D1)
  b2r = b2.reshape(1, D2)
  bgr = bg.reshape(1, 84)
  bfr = bf.reshape(1, 42)
  bor = bo.reshape(1, 1)

  Wop = jnp.pad(Wo, ((0, 7), (0, 0)))
  degparts = _deg_call(dst2, oneswd, zeroswd)
  hs_list = _tc1_call(x, W1p, degparts)
  msgsum1 = _mp1_call(src2, dst2, hs_list, zeroscw)
  hs2_list = _tc2_call(msgsum1, hs_list, degparts, b1r, W2)
  msgsum2 = _mp2_call(src2, dst2, hs2_list, zeroscw)
  return _tc3_call(msgsum2, hs2_list, degparts, b2r, batch2,
                   Wg, bgr, Wf, bfr, Wop, bor)


# speedup vs baseline: 2.8901x; 1.5287x over previous
import functools

import jax
import jax.numpy as jnp
from jax import lax
from jax.experimental import pallas as pl
from jax.experimental.pallas import tpu as pltpu
from jax.experimental.pallas import tpu_sc as plsc

N = 10000
E = 320000
G = 64
D1 = 336
D2 = 168
CW = 128
NCH1 = 3
NCH2 = 2
D1P = NCH1 * CW
D2P = NCH2 * CW
WD = 128
CHUNK = 128
NCHUNKS = E // CHUNK
NC = 2
NS = 16
TBLK = 256
NT = E // TBLK
GROUPS = N // 8
GPER, GREM = divmod(GROUPS, NS)
SGRP = 13
SROWS = SGRP * 8
RBLK = 1000
NBLK = N // RBLK

_mesh = functools.partial(
    plsc.VectorSubcoreMesh, core_axis_name="c", subcore_axis_name="s",
    num_cores=NC, num_subcores=NS)


def _my_chunk_range(sid, per_sc, sc_chunk0, nsplit=NS):
  per, rem = divmod(per_sc, nsplit)
  n_my = per + jnp.where(sid < rem, 1, 0)
  base = sc_chunk0 + sid * per + jnp.minimum(sid, rem)
  return base, n_my, per + (1 if rem else 0)


def _tile_rows(s):
  base_g = s * GPER + jnp.minimum(s, GREM)
  return base_g * 8


def _copy_rows_out(s, acc_s, stage_v, write_fn):
  r0 = _tile_rows(s)
  for k in range(GPER // SGRP):
    rr = pl.multiple_of(r0 + k * SROWS, 8)
    pltpu.sync_copy(acc_s.at[pl.ds(rr, SROWS)], stage_v)
    write_fn(stage_v, rr, SROWS)

  @pl.when(s < GREM)
  def _():
    rr = pl.multiple_of(r0 + GPER * 8, 8)
    pltpu.sync_copy(acc_s.at[pl.ds(rr, 8)], stage_v.at[pl.ds(0, 8)])
    write_fn(stage_v.at[pl.ds(0, 8)], rr, 8)


def _zero_rows(s, acc_s, zstage_v):
  r0 = _tile_rows(s)
  for k in range(GPER // SGRP):
    rr = pl.multiple_of(r0 + k * SROWS, 8)
    pltpu.sync_copy(zstage_v, acc_s.at[pl.ds(rr, SROWS)])

  @pl.when(s < GREM)
  def _():
    rr = pl.multiple_of(r0 + GPER * 8, 8)
    pltpu.sync_copy(zstage_v.at[pl.ds(0, 8)], acc_s.at[pl.ds(rr, 8)])


def _deg_body(dst2_hbm, ones_hbm, zeros_hbm, out_hbm,
              idx_v, ones_v, zstage_v, acc_s):
  c = lax.axis_index("c")
  s = lax.axis_index("s")
  wid = c * NS + s
  pltpu.sync_copy(ones_hbm, ones_v)
  pltpu.sync_copy(zeros_hbm, zstage_v)
  _zero_rows(s, acc_s, zstage_v)
  plsc.subcore_barrier()

  per, rem = divmod(NT, NC * NS)
  n_my = per + jnp.where(wid < rem, 1, 0)
  base = wid * per + jnp.minimum(wid, rem)

  def chunk_body(i, carry):
    @pl.when(i < n_my)
    def _():
      pltpu.sync_copy(dst2_hbm.at[base + i], idx_v)
      pltpu.sync_copy(ones_v, acc_s.at[idx_v], add=True)
    return carry

  lax.fori_loop(0, per + (1 if rem else 0), chunk_body, 0)
  plsc.subcore_barrier()

  def write_fn(stg, rr, nrows):
    pltpu.sync_copy(stg, out_hbm.at[c, pl.ds(rr, nrows)])

  _copy_rows_out(s, acc_s, zstage_v, write_fn)


def _deg_call(dst2, ones_arr, zeros_arr):
  return pl.kernel(
      _deg_body,
      out_type=jax.ShapeDtypeStruct((NC, N, WD), jnp.float32),
      mesh=_mesh(),
      scratch_types=[
          pltpu.VMEM((TBLK,), jnp.int32),
          pltpu.VMEM((TBLK, WD), jnp.float32),
          pltpu.VMEM((SROWS, WD), jnp.float32),
          pltpu.VMEM_SHARED((N, WD), jnp.float32),
      ],
      name="sc_deg_hist",
  )(dst2, ones_arr, zeros_arr)


def _mp_pipe(h_hbm, src2, dst2, acc_s, base, n_my, nmax, sidx, didx, rows):
  def t_body(i, carry):
    @pl.when(i < n_my)
    def _():
      pltpu.sync_copy(src2.at[base + i], sidx)
      pltpu.sync_copy(dst2.at[base + i], didx)
      pltpu.sync_copy(h_hbm.at[sidx], rows)
      pltpu.sync_copy(rows, acc_s.at[didx], add=True)
    return carry

  lax.fori_loop(0, nmax, t_body, 0)


def _mp_body(nsrc, hsel, split_fn, *refs):
  h_hbms = refs[:nsrc]
  src2, dst2, zeros_hbm, out_hbm = refs[nsrc:nsrc + 4]
  (sidx, didx, rows, zstage_v, acc_s) = refs[nsrc + 4:]
  c = lax.axis_index("c")
  s = lax.axis_index("s")
  pltpu.sync_copy(zeros_hbm, zstage_v)
  _zero_rows(s, acc_s, zstage_v)
  plsc.subcore_barrier()

  for cc in range(NC):
    @pl.when(c == cc)
    def _(cc=cc):
      base, n_my, nmax = split_fn(cc, s)
      _mp_pipe(h_hbms[hsel[cc]], src2, dst2, acc_s, base, n_my, nmax,
               sidx, didx, rows)

  plsc.subcore_barrier()

  def write_fn(stg, rr, nrows):
    pltpu.sync_copy(stg, out_hbm.at[c, pl.ds(rr, nrows)])

  _copy_rows_out(s, acc_s, zstage_v, write_fn)


def _mp_kernel(nsrc, hsel, split_fn, name, h_arrs, src2, dst2, zeros_arr):
  return pl.kernel(
      functools.partial(_mp_body, nsrc, hsel, split_fn),
      out_type=jax.ShapeDtypeStruct((NC, N, CW), jnp.float32),
      mesh=_mesh(),
      scratch_types=[
          pltpu.VMEM((TBLK,), jnp.int32),
          pltpu.VMEM((TBLK,), jnp.int32),
          pltpu.VMEM((TBLK, CW), jnp.float32),
          pltpu.VMEM((SROWS, CW), jnp.float32),
          pltpu.VMEM_SHARED((N, CW), jnp.float32),
      ],
      name=name,
  )(*h_arrs, src2, dst2, zeros_arr)


def _split_half(cc, s):
  per, rem = divmod(NT // NC, NS)
  n_my = per + jnp.where(s < rem, 1, 0)
  base = cc * (NT // NC) + s * per + jnp.minimum(s, rem)
  return base, n_my, per + (1 if rem else 0)


def _split_full(cc, s):
  per, rem = divmod(NT, NS)
  n_my = per + jnp.where(s < rem, 1, 0)
  base = s * per + jnp.minimum(s, rem)
  return base, n_my, per + (1 if rem else 0)


def _mp1_call(src2, dst2, xp, zeros_arr):
  return _mp_kernel(1, (0, 0), _split_half, "sc_mp1",
                    (xp,), src2, dst2, zeros_arr)


def _mp2_call(src2, dst2, hs_list, zeros_arr):
  return _mp_kernel(2, (0, 1), _split_full, "sc_mp2",
                    tuple(hs_list), src2, dst2, zeros_arr)


def _dis_block(dp_ref):
  deg = dp_ref[0, :, 0:1] + dp_ref[1, :, 0:1] + 1.0
  return lax.rsqrt(deg)


def _tc1_body(x_ref, dp_ref, xp_ref):
  dis = _dis_block(dp_ref)
  xp_ref[...] = x_ref[...] * dis


def _tc1_call(x, degparts):
  return pl.pallas_call(
      _tc1_body,
      grid=(NBLK,),
      in_specs=[
          pl.BlockSpec((RBLK, 128), lambda i: (i, 0)),
          pl.BlockSpec((NC, RBLK, WD), lambda i: (0, i, 0)),
      ],
      out_specs=pl.BlockSpec((RBLK, CW), lambda i: (i, 0)),
      out_shape=jax.ShapeDtypeStruct((N, CW), jnp.float32),
      name="tc1_scale",
  )(x, degparts)


def _tc2_body(ms_ref, xp_ref, dp_ref, b1_ref, w1_ref, w2_ref,
              hs2a_ref, hs2b_ref):
  dis = _dis_block(dp_ref)
  p = dis * (ms_ref[0] + ms_ref[1] + xp_ref[...])
  h1p = lax.dot_general(p, w1_ref[...], (((1,), (1,)), ((), ())),
                        preferred_element_type=jnp.float32)
  h1 = jax.nn.relu(h1p[:, :D1] + b1_ref[...])
  h2 = lax.dot_general(h1, w2_ref[...], (((1,), (1,)), ((), ())),
                       preferred_element_type=jnp.float32)
  hs2 = h2 * dis
  hs2a_ref[...] = hs2[:, :CW]
  hs2b_ref[...] = jnp.concatenate(
      [hs2[:, CW:], jnp.zeros((RBLK, D2P - D2), jnp.float32)], axis=1)


def _tc2_call(msgsum1, xp, degparts, b1r, W1p, W2):
  return pl.pallas_call(
      _tc2_body,
      grid=(NBLK,),
      in_specs=[
          pl.BlockSpec((NC, RBLK, CW), lambda i: (0, i, 0)),
          pl.BlockSpec((RBLK, CW), lambda i: (i, 0)),
          pl.BlockSpec((NC, RBLK, WD), lambda i: (0, i, 0)),
          pl.BlockSpec((1, D1), lambda i: (0, 0)),
          pl.BlockSpec((D1P, 128), lambda i: (0, 0)),
          pl.BlockSpec((D2, D1), lambda i: (0, 0)),
      ],
      out_specs=[
          pl.BlockSpec((RBLK, CW), lambda i: (i, 0)),
          pl.BlockSpec((RBLK, CW), lambda i: (i, 0)),
      ],
      out_shape=[
          jax.ShapeDtypeStruct((N, CW), jnp.float32),
          jax.ShapeDtypeStruct((N, CW), jnp.float32),
      ],
      name="tc2_gcn_matmuls",
  )(msgsum1, xp, degparts, b1r, W1p, W2)


def _tc3_body(ms_ref, hs2a_ref, hs2b_ref, dp_ref, b2_ref, batch_ref,
              wg_ref, bg_ref, wf_ref, bf_ref, wo_ref, bo_ref,
              o_ref, acc_ref):
  i = pl.program_id(0)

  @pl.when(i == 0)
  def _():
    acc_ref[...] = jnp.full((G, D2), -jnp.inf, dtype=jnp.float32)

  dis = _dis_block(dp_ref)
  msf = jnp.concatenate([ms_ref[0], ms_ref[1]], axis=1)[:, :D2]
  hsf = jnp.concatenate([hs2a_ref[...], hs2b_ref[...]], axis=1)[:, :D2]
  h = jax.nn.relu(dis * (msf + hsf) + b2_ref[...])
  b = batch_ref[...]
  glo = jnp.min(b)
  ghi = jnp.max(b)

  def seg_body(g, carry):
    m = (b == g)
    v = jnp.max(jnp.where(m, h, -jnp.inf), axis=0, keepdims=True)
    acc_ref[pl.ds(g, 1), :] = jnp.maximum(acc_ref[pl.ds(g, 1), :], v)
    return carry

  lax.fori_loop(glo, ghi + 1, seg_body, 0)

  @pl.when(i == NBLK - 1)
  def _():
    g0 = acc_ref[...]
    g1 = jax.nn.relu(
        lax.dot_general(g0, wg_ref[...], (((1,), (1,)), ((), ())),
                        preferred_element_type=jnp.float32) + bg_ref[...])
    g2 = jax.nn.relu(
        lax.dot_general(g1, wf_ref[...], (((1,), (1,)), ((), ())),
                        preferred_element_type=jnp.float32) + bf_ref[...])
    res = lax.dot_general(
        g2, wo_ref[...], (((1,), (1,)), ((), ())),
        preferred_element_type=jnp.float32)
    o_ref[...] = res[:, 0:1] + bo_ref[0, 0]


def _tc3_call(msgsum2, hs2_list, degparts, b2r, batch2,
              Wg, bgr, Wf, bfr, Wo, bor):
  return pl.pallas_call(
      _tc3_body,
      grid=(NBLK,),
      in_specs=[
          pl.BlockSpec((NCH2, RBLK, CW), lambda i: (0, i, 0)),
          pl.BlockSpec((RBLK, CW), lambda i: (i, 0)),
          pl.BlockSpec((RBLK, CW), lambda i: (i, 0)),
          pl.BlockSpec((NC, RBLK, WD), lambda i: (0, i, 0)),
          pl.BlockSpec((1, D2), lambda i: (0, 0)),
          pl.BlockSpec((RBLK, 1), lambda i: (i, 0)),
          pl.BlockSpec((84, D2), lambda i: (0, 0)),
          pl.BlockSpec((1, 84), lambda i: (0, 0)),
          pl.BlockSpec((42, 84), lambda i: (0, 0)),
          pl.BlockSpec((1, 42), lambda i: (0, 0)),
          pl.BlockSpec((8, 42), lambda i: (0, 0)),
          pl.BlockSpec((1, 1), lambda i: (0, 0)),
      ],
      out_specs=pl.BlockSpec((G, 1), lambda i: (0, 0)),
      out_shape=jax.ShapeDtypeStruct((G, 1), jnp.float32),
      scratch_shapes=[pltpu.VMEM((G, D2), jnp.float32)],
      name="tc3_pool_mlp",
  )(msgsum2, *hs2_list, degparts, b2r, batch2,
    Wg, bgr, Wf, bfr, Wo, bor)


def kernel(x, edge_index, batch, W1, b1, W2, b2, Wg, bg, Wf, bf, Wo, bo):
  src1d = edge_index[0]
  dst1d = edge_index[1]
  src2 = src1d.reshape(NT, TBLK)
  dst2 = dst1d.reshape(NT, TBLK)
  W1p = jnp.pad(W1, ((0, D1P - D1), (0, 0)))
  oneswd = jnp.ones((TBLK, WD), jnp.float32)
  zeroswd = jnp.zeros((SROWS, WD), jnp.float32)
  zeroscw = jnp.zeros((SROWS, CW), jnp.float32)
  batch2 = batch.reshape(N, 1)
  b1r = b1.reshape(1, D1)
  b2r = b2.reshape(1, D2)
  bgr = bg.reshape(1, 84)
  bfr = bf.reshape(1, 42)
  bor = bo.reshape(1, 1)

  Wop = jnp.pad(Wo, ((0, 7), (0, 0)))
  degparts = _deg_call(dst2, oneswd, zeroswd)
  xp = _tc1_call(x, degparts)
  msgsum1 = _mp1_call(src2, dst2, xp, zeroscw)
  hs2_list = _tc2_call(msgsum1, xp, degparts, b1r, W1p, W2)
  msgsum2 = _mp2_call(src2, dst2, hs2_list, zeroscw)
  return _tc3_call(msgsum2, hs2_list, degparts, b2r, batch2,
                   Wg, bgr, Wf, bfr, Wop, bor)


# speedup vs baseline: 3.4260x; 1.1854x over previous
import functools

import jax
import jax.numpy as jnp
from jax import lax
from jax.experimental import pallas as pl
from jax.experimental.pallas import tpu as pltpu
from jax.experimental.pallas import tpu_sc as plsc

N = 10000
E = 320000
G = 64
D1 = 336
D2 = 168
CW = 128
NCH1 = 3
NCH2 = 2
D1P = NCH1 * CW
D2P = NCH2 * CW
WD = 128
CHUNK = 128
NCHUNKS = E // CHUNK
NC = 2
NS = 16
TBLK = 256
NT = E // TBLK
GROUPS = N // 8
GPER, GREM = divmod(GROUPS, NS)
SGRP = 13
SROWS = SGRP * 8
RBLK = 1000
NBLK = N // RBLK

_mesh = functools.partial(
    plsc.VectorSubcoreMesh, core_axis_name="c", subcore_axis_name="s",
    num_cores=NC, num_subcores=NS)


def _my_chunk_range(sid, per_sc, sc_chunk0, nsplit=NS):
  per, rem = divmod(per_sc, nsplit)
  n_my = per + jnp.where(sid < rem, 1, 0)
  base = sc_chunk0 + sid * per + jnp.minimum(sid, rem)
  return base, n_my, per + (1 if rem else 0)


def _tile_rows(s):
  base_g = s * GPER + jnp.minimum(s, GREM)
  return base_g * 8


def _copy_rows_out(s, acc_s, stage_v, write_fn):
  r0 = _tile_rows(s)
  for k in range(GPER // SGRP):
    rr = pl.multiple_of(r0 + k * SROWS, 8)
    pltpu.sync_copy(acc_s.at[pl.ds(rr, SROWS)], stage_v)
    write_fn(stage_v, rr, SROWS)

  @pl.when(s < GREM)
  def _():
    rr = pl.multiple_of(r0 + GPER * 8, 8)
    pltpu.sync_copy(acc_s.at[pl.ds(rr, 8)], stage_v.at[pl.ds(0, 8)])
    write_fn(stage_v.at[pl.ds(0, 8)], rr, 8)


def _zero_rows(s, acc_s, zstage_v):
  r0 = _tile_rows(s)
  for k in range(GPER // SGRP):
    rr = pl.multiple_of(r0 + k * SROWS, 8)
    pltpu.sync_copy(zstage_v, acc_s.at[pl.ds(rr, SROWS)])

  @pl.when(s < GREM)
  def _():
    rr = pl.multiple_of(r0 + GPER * 8, 8)
    pltpu.sync_copy(zstage_v.at[pl.ds(0, 8)], acc_s.at[pl.ds(rr, 8)])


def _deg_body(dst2_hbm, ones_hbm, zeros_hbm, out_hbm,
              idx_v, ones_v, zstage_v, acc_s):
  c = lax.axis_index("c")
  s = lax.axis_index("s")
  wid = c * NS + s
  pltpu.sync_copy(ones_hbm, ones_v)
  pltpu.sync_copy(zeros_hbm, zstage_v)
  _zero_rows(s, acc_s, zstage_v)
  plsc.subcore_barrier()

  per, rem = divmod(NT, NC * NS)
  n_my = per + jnp.where(wid < rem, 1, 0)
  base = wid * per + jnp.minimum(wid, rem)

  def chunk_body(i, carry):
    @pl.when(i < n_my)
    def _():
      pltpu.sync_copy(dst2_hbm.at[base + i], idx_v)
      pltpu.sync_copy(ones_v, acc_s.at[idx_v], add=True)
    return carry

  lax.fori_loop(0, per + (1 if rem else 0), chunk_body, 0)
  plsc.subcore_barrier()

  def write_fn(stg, rr, nrows):
    pltpu.sync_copy(stg, out_hbm.at[c, pl.ds(rr, nrows)])

  _copy_rows_out(s, acc_s, zstage_v, write_fn)


def _deg_call(dst2, ones_arr, zeros_arr):
  return pl.kernel(
      _deg_body,
      out_type=jax.ShapeDtypeStruct((NC, N, WD), jnp.float32),
      mesh=_mesh(),
      scratch_types=[
          pltpu.VMEM((TBLK,), jnp.int32),
          pltpu.VMEM((TBLK, WD), jnp.float32),
          pltpu.VMEM((SROWS, WD), jnp.float32),
          pltpu.VMEM_SHARED((N, WD), jnp.float32),
      ],
      name="sc_deg_hist",
  )(dst2, ones_arr, zeros_arr)


def _mp_pipe(h_hbm, src1, dst1, acc_s, base, n_my, nmax, sidx, didx, rows,
             gsems, ssems):
  def gdesc(bb):
    return pltpu.make_async_copy(h_hbm.at[sidx.at[bb]], rows.at[bb],
                                 gsems[bb])

  def sdesc(bb):
    return pltpu.make_async_copy(rows.at[bb], acc_s.at[didx.at[bb]],
                                 ssems[bb])

  def step(i, carry):
    bb = lax.rem(i, 2)
    for b in (0, 1):
      @pl.when((bb == b) & (i >= 2) & (i < n_my))
      def _(b=b):
        sdesc(b).wait()

      @pl.when((bb == b) & (i < n_my))
      def _(b=b):
        off = pl.multiple_of((base + i) * CHUNK, CHUNK)
        pltpu.sync_copy(src1.at[pl.ds(off, CHUNK)], sidx.at[b])
        pltpu.sync_copy(dst1.at[pl.ds(off, CHUNK)], didx.at[b])
        gdesc(b).start()

      @pl.when((bb == b) & (i >= 1) & (i <= n_my))
      def _(b=b):
        gdesc(1 - b).wait()
        sdesc(1 - b).start(add=True)
    return carry

  lax.fori_loop(0, nmax + 1, step, 0)
  sdesc(0).wait()
  sdesc(1).wait()


def _mp_body(nsrc, hsel, split_fn, *refs):
  h_hbms = refs[:nsrc]
  src2, dst2, zeros_hbm, out_hbm = refs[nsrc:nsrc + 4]
  (sidx, didx, rows, zstage_v, acc_s, gs0, gs1, ss0, ss1) = refs[nsrc + 4:]
  gsems = (gs0, gs1)
  ssems = (ss0, ss1)
  c = lax.axis_index("c")
  s = lax.axis_index("s")
  pltpu.sync_copy(zeros_hbm, zstage_v)
  _zero_rows(s, acc_s, zstage_v)
  plsc.subcore_barrier()

  for cc in range(NC):
    @pl.when(c == cc)
    def _(cc=cc):
      base, n_my, nmax = split_fn(cc, s)
      _mp_pipe(h_hbms[hsel[cc]], src2, dst2, acc_s, base, n_my, nmax,
               sidx, didx, rows, gsems, ssems)

  plsc.subcore_barrier()

  def write_fn(stg, rr, nrows):
    pltpu.sync_copy(stg, out_hbm.at[c, pl.ds(rr, nrows)])

  _copy_rows_out(s, acc_s, zstage_v, write_fn)


def _mp_kernel(nsrc, hsel, split_fn, name, h_arrs, src2, dst2, zeros_arr):
  return pl.kernel(
      functools.partial(_mp_body, nsrc, hsel, split_fn),
      out_type=jax.ShapeDtypeStruct((NC, N, CW), jnp.float32),
      mesh=_mesh(),
      scratch_types=[
          pltpu.VMEM((2, CHUNK), jnp.int32),
          pltpu.VMEM((2, CHUNK), jnp.int32),
          pltpu.VMEM((2, CHUNK, CW), jnp.float32),
          pltpu.VMEM((SROWS, CW), jnp.float32),
          pltpu.VMEM_SHARED((N, CW), jnp.float32),
          pltpu.SemaphoreType.DMA,
          pltpu.SemaphoreType.DMA,
          pltpu.SemaphoreType.DMA,
          pltpu.SemaphoreType.DMA,
      ],
      name=name,
  )(*h_arrs, src2, dst2, zeros_arr)


def _split_half(cc, s):
  per, rem = divmod(NCHUNKS // NC, NS)
  n_my = per + jnp.where(s < rem, 1, 0)
  base = cc * (NCHUNKS // NC) + s * per + jnp.minimum(s, rem)
  return base, n_my, per + (1 if rem else 0)


def _split_full(cc, s):
  per, rem = divmod(NCHUNKS, NS)
  n_my = per + jnp.where(s < rem, 1, 0)
  base = s * per + jnp.minimum(s, rem)
  return base, n_my, per + (1 if rem else 0)


def _mp1_call(src2, dst2, xp, zeros_arr):
  return _mp_kernel(1, (0, 0), _split_half, "sc_mp1",
                    (xp,), src2, dst2, zeros_arr)


def _mp2_call(src2, dst2, hs_list, zeros_arr):
  return _mp_kernel(2, (0, 1), _split_full, "sc_mp2",
                    tuple(hs_list), src2, dst2, zeros_arr)


def _dis_block(dp_ref):
  deg = dp_ref[0, :, 0:1] + dp_ref[1, :, 0:1] + 1.0
  return lax.rsqrt(deg)


def _tc1_body(x_ref, dp_ref, xp_ref):
  dis = _dis_block(dp_ref)
  xp_ref[...] = x_ref[...] * dis


def _tc1_call(x, degparts):
  return pl.pallas_call(
      _tc1_body,
      grid=(NBLK,),
      in_specs=[
          pl.BlockSpec((RBLK, 128), lambda i: (i, 0)),
          pl.BlockSpec((NC, RBLK, WD), lambda i: (0, i, 0)),
      ],
      out_specs=pl.BlockSpec((RBLK, CW), lambda i: (i, 0)),
      out_shape=jax.ShapeDtypeStruct((N, CW), jnp.float32),
      name="tc1_scale",
  )(x, degparts)


def _tc2_body(ms_ref, xp_ref, dp_ref, b1_ref, w1_ref, w2_ref,
              hs2a_ref, hs2b_ref):
  dis = _dis_block(dp_ref)
  p = dis * (ms_ref[0] + ms_ref[1] + xp_ref[...])
  h1p = lax.dot_general(p, w1_ref[...], (((1,), (1,)), ((), ())),
                        preferred_element_type=jnp.float32)
  h1 = jax.nn.relu(h1p[:, :D1] + b1_ref[...])
  h2 = lax.dot_general(h1, w2_ref[...], (((1,), (1,)), ((), ())),
                       preferred_element_type=jnp.float32)
  hs2 = h2 * dis
  hs2a_ref[...] = hs2[:, :CW]
  hs2b_ref[...] = jnp.concatenate(
      [hs2[:, CW:], jnp.zeros((RBLK, D2P - D2), jnp.float32)], axis=1)


def _tc2_call(msgsum1, xp, degparts, b1r, W1p, W2):
  return pl.pallas_call(
      _tc2_body,
      grid=(NBLK,),
      in_specs=[
          pl.BlockSpec((NC, RBLK, CW), lambda i: (0, i, 0)),
          pl.BlockSpec((RBLK, CW), lambda i: (i, 0)),
          pl.BlockSpec((NC, RBLK, WD), lambda i: (0, i, 0)),
          pl.BlockSpec((1, D1), lambda i: (0, 0)),
          pl.BlockSpec((D1P, 128), lambda i: (0, 0)),
          pl.BlockSpec((D2, D1), lambda i: (0, 0)),
      ],
      out_specs=[
          pl.BlockSpec((RBLK, CW), lambda i: (i, 0)),
          pl.BlockSpec((RBLK, CW), lambda i: (i, 0)),
      ],
      out_shape=[
          jax.ShapeDtypeStruct((N, CW), jnp.float32),
          jax.ShapeDtypeStruct((N, CW), jnp.float32),
      ],
      name="tc2_gcn_matmuls",
  )(msgsum1, xp, degparts, b1r, W1p, W2)


def _tc3_body(ms_ref, hs2a_ref, hs2b_ref, dp_ref, b2_ref, batch_ref,
              wg_ref, bg_ref, wf_ref, bf_ref, wo_ref, bo_ref,
              o_ref, acc_ref):
  i = pl.program_id(0)

  @pl.when(i == 0)
  def _():
    acc_ref[...] = jnp.full((G, D2), -jnp.inf, dtype=jnp.float32)

  dis = _dis_block(dp_ref)
  msf = jnp.concatenate([ms_ref[0], ms_ref[1]], axis=1)[:, :D2]
  hsf = jnp.concatenate([hs2a_ref[...], hs2b_ref[...]], axis=1)[:, :D2]
  h = jax.nn.relu(dis * (msf + hsf) + b2_ref[...])
  b = batch_ref[...]
  glo = jnp.min(b)
  ghi = jnp.max(b)

  def seg_body(g, carry):
    m = (b == g)
    v = jnp.max(jnp.where(m, h, -jnp.inf), axis=0, keepdims=True)
    acc_ref[pl.ds(g, 1), :] = jnp.maximum(acc_ref[pl.ds(g, 1), :], v)
    return carry

  lax.fori_loop(glo, ghi + 1, seg_body, 0)

  @pl.when(i == NBLK - 1)
  def _():
    g0 = acc_ref[...]
    g1 = jax.nn.relu(
        lax.dot_general(g0, wg_ref[...], (((1,), (1,)), ((), ())),
                        preferred_element_type=jnp.float32) + bg_ref[...])
    g2 = jax.nn.relu(
        lax.dot_general(g1, wf_ref[...], (((1,), (1,)), ((), ())),
                        preferred_element_type=jnp.float32) + bf_ref[...])
    res = lax.dot_general(
        g2, wo_ref[...], (((1,), (1,)), ((), ())),
        preferred_element_type=jnp.float32)
    o_ref[...] = res[:, 0:1] + bo_ref[0, 0]


def _tc3_call(msgsum2, hs2_list, degparts, b2r, batch2,
              Wg, bgr, Wf, bfr, Wo, bor):
  return pl.pallas_call(
      _tc3_body,
      grid=(NBLK,),
      in_specs=[
          pl.BlockSpec((NCH2, RBLK, CW), lambda i: (0, i, 0)),
          pl.BlockSpec((RBLK, CW), lambda i: (i, 0)),
          pl.BlockSpec((RBLK, CW), lambda i: (i, 0)),
          pl.BlockSpec((NC, RBLK, WD), lambda i: (0, i, 0)),
          pl.BlockSpec((1, D2), lambda i: (0, 0)),
          pl.BlockSpec((RBLK, 1), lambda i: (i, 0)),
          pl.BlockSpec((84, D2), lambda i: (0, 0)),
          pl.BlockSpec((1, 84), lambda i: (0, 0)),
          pl.BlockSpec((42, 84), lambda i: (0, 0)),
          pl.BlockSpec((1, 42), lambda i: (0, 0)),
          pl.BlockSpec((8, 42), lambda i: (0, 0)),
          pl.BlockSpec((1, 1), lambda i: (0, 0)),
      ],
      out_specs=pl.BlockSpec((G, 1), lambda i: (0, 0)),
      out_shape=jax.ShapeDtypeStruct((G, 1), jnp.float32),
      scratch_shapes=[pltpu.VMEM((G, D2), jnp.float32)],
      name="tc3_pool_mlp",
  )(msgsum2, *hs2_list, degparts, b2r, batch2,
    Wg, bgr, Wf, bfr, Wo, bor)


def kernel(x, edge_index, batch, W1, b1, W2, b2, Wg, bg, Wf, bf, Wo, bo):
  src1d = edge_index[0]
  dst1d = edge_index[1]
  src2 = src1d.reshape(NT, TBLK)
  dst2 = dst1d.reshape(NT, TBLK)
  W1p = jnp.pad(W1, ((0, D1P - D1), (0, 0)))
  oneswd = jnp.ones((TBLK, WD), jnp.float32)
  zeroswd = jnp.zeros((SROWS, WD), jnp.float32)
  zeroscw = jnp.zeros((SROWS, CW), jnp.float32)
  batch2 = batch.reshape(N, 1)
  b1r = b1.reshape(1, D1)
  b2r = b2.reshape(1, D2)
  bgr = bg.reshape(1, 84)
  bfr = bf.reshape(1, 42)
  bor = bo.reshape(1, 1)

  Wop = jnp.pad(Wo, ((0, 7), (0, 0)))
  degparts = _deg_call(dst2, oneswd, zeroswd)
  xp = _tc1_call(x, degparts)
  msgsum1 = _mp1_call(src1d, dst1d, xp, zeroscw)
  hs2_list = _tc2_call(msgsum1, xp, degparts, b1r, W1p, W2)
  msgsum2 = _mp2_call(src1d, dst1d, hs2_list, zeroscw)
  return _tc3_call(msgsum2, hs2_list, degparts, b2r, batch2,
                   Wg, bgr, Wf, bfr, Wop, bor)


# speedup vs baseline: 3.5134x; 1.0255x over previous
import functools

import jax
import jax.numpy as jnp
from jax import lax
from jax.experimental import pallas as pl
from jax.experimental.pallas import tpu as pltpu
from jax.experimental.pallas import tpu_sc as plsc

N = 10000
E = 320000
G = 64
D1 = 336
D2 = 168
CW = 128
NCH1 = 3
NCH2 = 2
D1P = NCH1 * CW
D2P = NCH2 * CW
WD = 128
CHUNK = 128
NCHUNKS = E // CHUNK
NC = 2
NS = 16
TBLK = 256
NT = E // TBLK
GROUPS = N // 8
GPER, GREM = divmod(GROUPS, NS)
SGRP = 13
SROWS = SGRP * 8
RBLK = 1000
NBLK = N // RBLK

_mesh = functools.partial(
    plsc.VectorSubcoreMesh, core_axis_name="c", subcore_axis_name="s",
    num_cores=NC, num_subcores=NS)


def _my_chunk_range(sid, per_sc, sc_chunk0, nsplit=NS):
  per, rem = divmod(per_sc, nsplit)
  n_my = per + jnp.where(sid < rem, 1, 0)
  base = sc_chunk0 + sid * per + jnp.minimum(sid, rem)
  return base, n_my, per + (1 if rem else 0)


def _tile_rows(s):
  base_g = s * GPER + jnp.minimum(s, GREM)
  return base_g * 8


def _copy_rows_out(s, acc_s, stage_v, write_fn):
  r0 = _tile_rows(s)
  for k in range(GPER // SGRP):
    rr = pl.multiple_of(r0 + k * SROWS, 8)
    pltpu.sync_copy(acc_s.at[pl.ds(rr, SROWS)], stage_v)
    write_fn(stage_v, rr, SROWS)

  @pl.when(s < GREM)
  def _():
    rr = pl.multiple_of(r0 + GPER * 8, 8)
    pltpu.sync_copy(acc_s.at[pl.ds(rr, 8)], stage_v.at[pl.ds(0, 8)])
    write_fn(stage_v.at[pl.ds(0, 8)], rr, 8)


def _zero_rows(s, acc_s, zstage_v):
  r0 = _tile_rows(s)
  for k in range(GPER // SGRP):
    rr = pl.multiple_of(r0 + k * SROWS, 8)
    pltpu.sync_copy(zstage_v, acc_s.at[pl.ds(rr, SROWS)])

  @pl.when(s < GREM)
  def _():
    rr = pl.multiple_of(r0 + GPER * 8, 8)
    pltpu.sync_copy(zstage_v.at[pl.ds(0, 8)], acc_s.at[pl.ds(rr, 8)])


def _deg_body(dst1_hbm, ones_hbm, zeros_hbm, out_hbm,
              didx, ones_v, zstage_v, acc_s, ss0, ss1):
  c = lax.axis_index("c")
  s = lax.axis_index("s")
  wid = c * NS + s
  ssems = (ss0, ss1)
  pltpu.sync_copy(ones_hbm, ones_v)
  pltpu.sync_copy(zeros_hbm, zstage_v)
  _zero_rows(s, acc_s, zstage_v)
  plsc.subcore_barrier()

  per, rem = divmod(NT, NC * NS)
  n_my = per + jnp.where(wid < rem, 1, 0)
  base = wid * per + jnp.minimum(wid, rem)

  def sdesc(bb):
    return pltpu.make_async_copy(
        ones_v, acc_s.at[didx.at[pl.ds(bb * TBLK, TBLK)]], ssems[bb])

  def step(i, carry):
    bb = lax.rem(i, 2)
    for b in (0, 1):
      @pl.when((bb == b) & (i >= 2) & (i < n_my))
      def _(b=b):
        sdesc(b).wait()

      @pl.when((bb == b) & (i < n_my))
      def _(b=b):
        off = pl.multiple_of((base + i) * TBLK, TBLK)
        pltpu.sync_copy(dst1_hbm.at[pl.ds(off, TBLK)],
                        didx.at[pl.ds(b * TBLK, TBLK)])
        sdesc(b).start(add=True)
    return carry

  lax.fori_loop(0, per + (1 if rem else 0), step, 0)
  sdesc(0).wait()
  sdesc(1).wait()
  plsc.subcore_barrier()

  def write_fn(stg, rr, nrows):
    pltpu.sync_copy(stg, out_hbm.at[c, pl.ds(rr, nrows)])

  _copy_rows_out(s, acc_s, zstage_v, write_fn)


def _deg_call(dst1, ones_arr, zeros_arr):
  return pl.kernel(
      _deg_body,
      out_type=jax.ShapeDtypeStruct((NC, N, WD), jnp.float32),
      mesh=_mesh(),
      scratch_types=[
          pltpu.VMEM((2 * TBLK,), jnp.int32),
          pltpu.VMEM((TBLK, WD), jnp.float32),
          pltpu.VMEM((SROWS, WD), jnp.float32),
          pltpu.VMEM_SHARED((N, WD), jnp.float32),
          pltpu.SemaphoreType.DMA,
          pltpu.SemaphoreType.DMA,
      ],
      name="sc_deg_hist",
  )(dst1, ones_arr, zeros_arr)


def _mp_pipe(h_hbm, src1, dst1, acc_s, base, n_my, nmax, sidx, didx, rows,
             gsems, ssems):
  def gdesc(bb):
    return pltpu.make_async_copy(h_hbm.at[sidx.at[bb]], rows.at[bb],
                                 gsems[bb])

  def sdesc(bb):
    return pltpu.make_async_copy(rows.at[bb], acc_s.at[didx.at[bb]],
                                 ssems[bb])

  def step(i, carry):
    bb = lax.rem(i, 2)
    for b in (0, 1):
      @pl.when((bb == b) & (i >= 2) & (i < n_my))
      def _(b=b):
        sdesc(b).wait()

      @pl.when((bb == b) & (i < n_my))
      def _(b=b):
        off = pl.multiple_of((base + i) * CHUNK, CHUNK)
        pltpu.sync_copy(src1.at[pl.ds(off, CHUNK)], sidx.at[b])
        pltpu.sync_copy(dst1.at[pl.ds(off, CHUNK)], didx.at[b])
        gdesc(b).start()

      @pl.when((bb == b) & (i >= 1) & (i <= n_my))
      def _(b=b):
        gdesc(1 - b).wait()
        sdesc(1 - b).start(add=True)
    return carry

  lax.fori_loop(0, nmax + 1, step, 0)
  sdesc(0).wait()
  sdesc(1).wait()


def _mp_body(nsrc, hsel, split_fn, *refs):
  h_hbms = refs[:nsrc]
  src2, dst2, zeros_hbm, out_hbm = refs[nsrc:nsrc + 4]
  (sidx, didx, rows, zstage_v, acc_s, gs0, gs1, ss0, ss1) = refs[nsrc + 4:]
  gsems = (gs0, gs1)
  ssems = (ss0, ss1)
  c = lax.axis_index("c")
  s = lax.axis_index("s")
  pltpu.sync_copy(zeros_hbm, zstage_v)
  _zero_rows(s, acc_s, zstage_v)
  plsc.subcore_barrier()

  for cc in range(NC):
    @pl.when(c == cc)
    def _(cc=cc):
      base, n_my, nmax = split_fn(cc, s)
      _mp_pipe(h_hbms[hsel[cc]], src2, dst2, acc_s, base, n_my, nmax,
               sidx, didx, rows, gsems, ssems)

  plsc.subcore_barrier()

  def write_fn(stg, rr, nrows):
    pltpu.sync_copy(stg, out_hbm.at[c, pl.ds(rr, nrows)])

  _copy_rows_out(s, acc_s, zstage_v, write_fn)


def _mp_kernel(nsrc, hsel, split_fn, name, h_arrs, src2, dst2, zeros_arr):
  return pl.kernel(
      functools.partial(_mp_body, nsrc, hsel, split_fn),
      out_type=jax.ShapeDtypeStruct((NC, N, CW), jnp.float32),
      mesh=_mesh(),
      scratch_types=[
          pltpu.VMEM((2, CHUNK), jnp.int32),
          pltpu.VMEM((2, CHUNK), jnp.int32),
          pltpu.VMEM((2, CHUNK, CW), jnp.float32),
          pltpu.VMEM((SROWS, CW), jnp.float32),
          pltpu.VMEM_SHARED((N, CW), jnp.float32),
          pltpu.SemaphoreType.DMA,
          pltpu.SemaphoreType.DMA,
          pltpu.SemaphoreType.DMA,
          pltpu.SemaphoreType.DMA,
      ],
      name=name,
  )(*h_arrs, src2, dst2, zeros_arr)


def _split_half(cc, s):
  per, rem = divmod(NCHUNKS // NC, NS)
  n_my = per + jnp.where(s < rem, 1, 0)
  base = cc * (NCHUNKS // NC) + s * per + jnp.minimum(s, rem)
  return base, n_my, per + (1 if rem else 0)


def _split_full(cc, s):
  per, rem = divmod(NCHUNKS, NS)
  n_my = per + jnp.where(s < rem, 1, 0)
  base = s * per + jnp.minimum(s, rem)
  return base, n_my, per + (1 if rem else 0)


def _mp1_call(src2, dst2, xp, zeros_arr):
  return _mp_kernel(1, (0, 0), _split_half, "sc_mp1",
                    (xp,), src2, dst2, zeros_arr)


def _mp2_call(src2, dst2, hs_list, zeros_arr):
  return _mp_kernel(2, (0, 1), _split_full, "sc_mp2",
                    tuple(hs_list), src2, dst2, zeros_arr)


def _dis_block(dp_ref):
  deg = dp_ref[0, :, 0:1] + dp_ref[1, :, 0:1] + 1.0
  return lax.rsqrt(deg)


def _tc1_body(x_ref, dp_ref, xp_ref):
  dis = _dis_block(dp_ref)
  xp_ref[...] = x_ref[...] * dis


def _tc1_call(x, degparts):
  return pl.pallas_call(
      _tc1_body,
      grid=(NBLK,),
      in_specs=[
          pl.BlockSpec((RBLK, 128), lambda i: (i, 0)),
          pl.BlockSpec((NC, RBLK, WD), lambda i: (0, i, 0)),
      ],
      out_specs=pl.BlockSpec((RBLK, CW), lambda i: (i, 0)),
      out_shape=jax.ShapeDtypeStruct((N, CW), jnp.float32),
      name="tc1_scale",
  )(x, degparts)


def _tc2_body(ms_ref, xp_ref, dp_ref, b1_ref, w1_ref, w2_ref,
              hs2a_ref, hs2b_ref):
  dis = _dis_block(dp_ref)
  p = dis * (ms_ref[0] + ms_ref[1] + xp_ref[...])
  h1p = lax.dot_general(p, w1_ref[...], (((1,), (1,)), ((), ())),
                        preferred_element_type=jnp.float32)
  h1 = jax.nn.relu(h1p[:, :D1] + b1_ref[...])
  h2 = lax.dot_general(h1, w2_ref[...], (((1,), (1,)), ((), ())),
                       preferred_element_type=jnp.float32)
  hs2 = h2 * dis
  hs2a_ref[...] = hs2[:, :CW]
  hs2b_ref[...] = jnp.concatenate(
      [hs2[:, CW:], jnp.zeros((RBLK, D2P - D2), jnp.float32)], axis=1)


def _tc2_call(msgsum1, xp, degparts, b1r, W1p, W2):
  return pl.pallas_call(
      _tc2_body,
      grid=(NBLK,),
      in_specs=[
          pl.BlockSpec((NC, RBLK, CW), lambda i: (0, i, 0)),
          pl.BlockSpec((RBLK, CW), lambda i: (i, 0)),
          pl.BlockSpec((NC, RBLK, WD), lambda i: (0, i, 0)),
          pl.BlockSpec((1, D1), lambda i: (0, 0)),
          pl.BlockSpec((D1P, 128), lambda i: (0, 0)),
          pl.BlockSpec((D2, D1), lambda i: (0, 0)),
      ],
      out_specs=[
          pl.BlockSpec((RBLK, CW), lambda i: (i, 0)),
          pl.BlockSpec((RBLK, CW), lambda i: (i, 0)),
      ],
      out_shape=[
          jax.ShapeDtypeStruct((N, CW), jnp.float32),
          jax.ShapeDtypeStruct((N, CW), jnp.float32),
      ],
      name="tc2_gcn_matmuls",
  )(msgsum1, xp, degparts, b1r, W1p, W2)


def _tc3_body(ms_ref, hs2a_ref, hs2b_ref, dp_ref, b2_ref, batch_ref,
              wg_ref, bg_ref, wf_ref, bf_ref, wo_ref, bo_ref,
              o_ref, acc_ref):
  i = pl.program_id(0)

  @pl.when(i == 0)
  def _():
    acc_ref[...] = jnp.full((G, D2), -jnp.inf, dtype=jnp.float32)

  dis = _dis_block(dp_ref)
  msf = jnp.concatenate([ms_ref[0], ms_ref[1]], axis=1)[:, :D2]
  hsf = jnp.concatenate([hs2a_ref[...], hs2b_ref[...]], axis=1)[:, :D2]
  h = jax.nn.relu(dis * (msf + hsf) + b2_ref[...])
  b = batch_ref[...]
  glo = jnp.min(b)
  ghi = jnp.max(b)

  def seg_body(g, carry):
    m = (b == g)
    v = jnp.max(jnp.where(m, h, -jnp.inf), axis=0, keepdims=True)
    acc_ref[pl.ds(g, 1), :] = jnp.maximum(acc_ref[pl.ds(g, 1), :], v)
    return carry

  lax.fori_loop(glo, ghi + 1, seg_body, 0)

  @pl.when(i == NBLK - 1)
  def _():
    g0 = acc_ref[...]
    g1 = jax.nn.relu(
        lax.dot_general(g0, wg_ref[...], (((1,), (1,)), ((), ())),
                        preferred_element_type=jnp.float32) + bg_ref[...])
    g2 = jax.nn.relu(
        lax.dot_general(g1, wf_ref[...], (((1,), (1,)), ((), ())),
                        preferred_element_type=jnp.float32) + bf_ref[...])
    res = lax.dot_general(
        g2, wo_ref[...], (((1,), (1,)), ((), ())),
        preferred_element_type=jnp.float32)
    o_ref[...] = res[:, 0:1] + bo_ref[0, 0]


def _tc3_call(msgsum2, hs2_list, degparts, b2r, batch2,
              Wg, bgr, Wf, bfr, Wo, bor):
  return pl.pallas_call(
      _tc3_body,
      grid=(NBLK,),
      in_specs=[
          pl.BlockSpec((NCH2, RBLK, CW), lambda i: (0, i, 0)),
          pl.BlockSpec((RBLK, CW), lambda i: (i, 0)),
          pl.BlockSpec((RBLK, CW), lambda i: (i, 0)),
          pl.BlockSpec((NC, RBLK, WD), lambda i: (0, i, 0)),
          pl.BlockSpec((1, D2), lambda i: (0, 0)),
          pl.BlockSpec((RBLK, 1), lambda i: (i, 0)),
          pl.BlockSpec((84, D2), lambda i: (0, 0)),
          pl.BlockSpec((1, 84), lambda i: (0, 0)),
          pl.BlockSpec((42, 84), lambda i: (0, 0)),
          pl.BlockSpec((1, 42), lambda i: (0, 0)),
          pl.BlockSpec((8, 42), lambda i: (0, 0)),
          pl.BlockSpec((1, 1), lambda i: (0, 0)),
      ],
      out_specs=pl.BlockSpec((G, 1), lambda i: (0, 0)),
      out_shape=jax.ShapeDtypeStruct((G, 1), jnp.float32),
      scratch_shapes=[pltpu.VMEM((G, D2), jnp.float32)],
      name="tc3_pool_mlp",
  )(msgsum2, *hs2_list, degparts, b2r, batch2,
    Wg, bgr, Wf, bfr, Wo, bor)


def kernel(x, edge_index, batch, W1, b1, W2, b2, Wg, bg, Wf, bf, Wo, bo):
  src1d = edge_index[0]
  dst1d = edge_index[1]
  W1p = jnp.pad(W1, ((0, D1P - D1), (0, 0)))
  oneswd = jnp.ones((TBLK, WD), jnp.float32)
  zeroswd = jnp.zeros((SROWS, WD), jnp.float32)
  zeroscw = jnp.zeros((SROWS, CW), jnp.float32)
  batch2 = batch.reshape(N, 1)
  b1r = b1.reshape(1, D1)
  b2r = b2.reshape(1, D2)
  bgr = bg.reshape(1, 84)
  bfr = bf.reshape(1, 42)
  bor = bo.reshape(1, 1)

  Wop = jnp.pad(Wo, ((0, 7), (0, 0)))
  degparts = _deg_call(dst1d, oneswd, zeroswd)
  xp = _tc1_call(x, degparts)
  msgsum1 = _mp1_call(src1d, dst1d, xp, zeroscw)
  hs2_list = _tc2_call(msgsum1, xp, degparts, b1r, W1p, W2)
  msgsum2 = _mp2_call(src1d, dst1d, hs2_list, zeroscw)
  return _tc3_call(msgsum2, hs2_list, degparts, b2r, batch2,
                   Wg, bgr, Wf, bfr, Wop, bor)
